# baseline (speedup 1.0000x reference)
import jax
import jax.numpy as jnp
from jax import lax
from jax.experimental import pallas as pl
from jax.experimental.pallas import tpu as pltpu
from jax.experimental.pallas import tpu_sc as plsc

_NUM_USERS = 500000
_LANES = 16


def _make_body(batch, emb, nw):
    bpw = batch // nw
    ch = 128
    nch = bpw // ch
    gpc = ch // _LANES

    def body(id_hbm, eu_hbm, ei_hbm, out_hbm, ids_v, uidx_v, iidx_v, gcomb,
             obuf, gsem, osem):
        nc = lax.axis_size("c")
        wid = lax.axis_index("s") * nc + lax.axis_index("c")
        base = wid * bpw
        iota = lax.iota(jnp.int32, _LANES)

        pltpu.sync_copy(id_hbm.at[pl.ds(base, bpw)], ids_v)

        def compute_idx(c):
            for k in range(gpc):
                idv = ids_v[pl.ds(c * ch + k * _LANES, _LANES)]
                m = idv < _NUM_USERS
                uidx_v[c, pl.ds(k * _LANES, _LANES)] = (
                    jnp.where(m, idv, 0) >> 1)
                iidx_v[c, pl.ds(k * _LANES, _LANES)] = (
                    jnp.where(m, 0, idv - _NUM_USERS) >> 1)

        def fire(c):
            p = c % 2
            return (
                pltpu.async_copy(eu_hbm.at[uidx_v.at[c]],
                                 gcomb.at[p, pl.ds(0, ch)], gsem),
                pltpu.async_copy(ei_hbm.at[iidx_v.at[c]],
                                 gcomb.at[p, pl.ds(ch, ch)], gsem),
            )

        compute_idx(0)
        handles = {0: fire(0)}
        ohandles = {}
        for c in range(nch):
            p = c % 2
            if c + 1 < nch:
                compute_idx(c + 1)
                handles[c + 1] = fire(c + 1)
            hu, hi = handles.pop(c)
            hu.wait()
            hi.wait()
            if c >= 2:
                ohandles.pop(c - 2).wait()

            def copy16(g, carry):
                idv = ids_v[pl.ds(c * ch + g * _LANES, _LANES)]
                m = idv < _NUM_USERS
                eff = jnp.where(m, idv, idv - _NUM_USERS)
                selr = g * _LANES + iota + jnp.where(m, 0, ch)
                colo = (eff & 1) * emb
                for u in range(_LANES):
                    sr = selr[u]
                    co = colo[u]
                    jrow = g * _LANES + u
                    for cc in range(emb // _LANES):
                        obuf[p, jrow, pl.ds(cc * _LANES, _LANES)] = (
                            gcomb[p, sr, pl.ds(co + cc * _LANES, _LANES)])
                return carry

            lax.fori_loop(0, gpc, copy16, 0)
            ohandles[c] = pltpu.async_copy(
                obuf.at[p], out_hbm.at[pl.ds(base + c * ch, ch)], osem)
        for c in sorted(ohandles):
            ohandles.pop(c).wait()

    return body, bpw, ch, nch


def kernel(id, e_user, e_item):
    batch = id.shape[0]
    emb = e_user.shape[1]
    nu = e_user.shape[0]
    ni = e_item.shape[0]
    eu2 = e_user.reshape(nu // 2, 2 * emb)
    ei2 = e_item.reshape(ni // 2, 2 * emb)
    info = plsc.get_sparse_core_info()
    nw = info.num_cores * info.num_subcores
    body, bpw, ch, nch = _make_body(batch, emb, nw)
    mesh = plsc.VectorSubcoreMesh(core_axis_name="c", subcore_axis_name="s")
    f = pl.kernel(
        body,
        out_type=jax.ShapeDtypeStruct((batch, emb), jnp.float32),
        mesh=mesh,
        compiler_params=pltpu.CompilerParams(use_tc_tiling_on_sc=True),
        scratch_types=[
            pltpu.VMEM((bpw,), jnp.int32),
            pltpu.VMEM((nch, ch), jnp.int32),
            pltpu.VMEM((nch, ch), jnp.int32),
            pltpu.VMEM((2, 2 * ch, 2 * emb), jnp.float32),
            pltpu.VMEM((2, ch, emb), jnp.float32),
            pltpu.SemaphoreType.DMA,
            pltpu.SemaphoreType.DMA,
        ],
    )
    return f(id, eu2, ei2)


# speedup vs baseline: 2.3940x; 2.3940x over previous
import jax
import jax.numpy as jnp
from jax import lax
from jax.experimental import pallas as pl
from jax.experimental.pallas import tpu as pltpu
from jax.experimental.pallas import tpu_sc as plsc

_NUM_USERS = 500000
_LANES = 16


def _make_body(batch, emb, nw):
    bpw = batch // nw
    ngrp = bpw // _LANES

    def body(id_hbm, eu_hbm, ei_hbm, out_hbm, ids_v, tbuf, obuf, gsem, osem):
        nc = lax.axis_size("c")
        wid = lax.axis_index("s") * nc + lax.axis_index("c")
        base = wid * bpw

        pltpu.sync_copy(id_hbm.at[pl.ds(base, bpw)], ids_v)

        def fire(g):
            p = g % 2
            idv = ids_v[pl.ds(g * _LANES, _LANES)]
            eff = jnp.where(idv < _NUM_USERS, idv, idv - _NUM_USERS)
            for u in range(_LANES):
                s = idv[u]
                t8 = pl.multiple_of((eff[u] >> 3) << 3, 8)

                @pl.when(s < _NUM_USERS)
                def _():
                    pltpu.async_copy(eu_hbm.at[pl.ds(t8, 8), :],
                                     tbuf.at[p, u], gsem)

                @pl.when(s >= _NUM_USERS)
                def _():
                    pltpu.async_copy(ei_hbm.at[pl.ds(t8, 8), :],
                                     tbuf.at[p, u], gsem)

        def drain_select(g):
            p = g % 2
            for u in range(_LANES):
                pltpu.make_async_copy(eu_hbm.at[pl.ds(0, 8), :],
                                      tbuf.at[p, u], gsem).wait()
            idv = ids_v[pl.ds(g * _LANES, _LANES)]
            eff = jnp.where(idv < _NUM_USERS, idv, idv - _NUM_USERS)
            sub = eff & 7
            for u in range(_LANES):
                sb = sub[u]
                r = g * _LANES + u
                for cc in range(emb // _LANES):
                    obuf[r, pl.ds(cc * _LANES, _LANES)] = (
                        tbuf[p, u, sb, pl.ds(cc * _LANES, _LANES)])

        fire(0)

        def pipe(g, carry):
            fire(g + 1)
            drain_select(g)
            return carry

        lax.fori_loop(0, ngrp - 1, pipe, 0)
        drain_select(ngrp - 1)
        pltpu.sync_copy(obuf, out_hbm.at[pl.ds(base, bpw)])

    return body, bpw


def kernel(id, e_user, e_item):
    batch = id.shape[0]
    emb = e_user.shape[1]
    info = plsc.get_sparse_core_info()
    nw = info.num_cores * info.num_subcores
    body, bpw = _make_body(batch, emb, nw)
    mesh = plsc.VectorSubcoreMesh(core_axis_name="c", subcore_axis_name="s")
    f = pl.kernel(
        body,
        out_type=jax.ShapeDtypeStruct((batch, emb), jnp.float32),
        mesh=mesh,
        compiler_params=pltpu.CompilerParams(use_tc_tiling_on_sc=True),
        scratch_types=[
            pltpu.VMEM((bpw,), jnp.int32),
            pltpu.VMEM((2, _LANES, 8, emb), jnp.float32),
            pltpu.VMEM((bpw, emb), jnp.float32),
            pltpu.SemaphoreType.DMA,
            pltpu.SemaphoreType.DMA,
        ],
    )
    return f(id, e_user, e_item)


# speedup vs baseline: 3.4004x; 1.4204x over previous
import jax
import jax.numpy as jnp
from jax import lax
from jax.experimental import pallas as pl
from jax.experimental.pallas import tpu as pltpu
from jax.experimental.pallas import tpu_sc as plsc

_NUM_USERS = 500000
_LANES = 16


def _make_body(batch, emb, nw):
    bpw = batch // nw
    ngrp = bpw // _LANES

    def body(id_hbm, eu_hbm, ei_hbm, out_hbm, ids_v, tbuf, obuf, gsem, osem):
        nc = lax.axis_size("c")
        wid = lax.axis_index("s") * nc + lax.axis_index("c")
        base = wid * bpw

        pltpu.sync_copy(id_hbm.at[pl.ds(base, bpw)], ids_v)

        def fire(g):
            p = g % 2
            idv = ids_v[pl.ds(g * _LANES, _LANES)]
            eff = jnp.where(idv < _NUM_USERS, idv, idv - _NUM_USERS)
            for u in range(_LANES):
                s = idv[u]
                t = eff[u] >> 3

                @pl.when(s < _NUM_USERS)
                def _():
                    pltpu.async_copy(eu_hbm.at[t], tbuf.at[p, u], gsem)

                @pl.when(s >= _NUM_USERS)
                def _():
                    pltpu.async_copy(ei_hbm.at[t], tbuf.at[p, u], gsem)

        def drain_select(g):
            p = g % 2
            for u in range(_LANES):
                pltpu.make_async_copy(eu_hbm.at[0],
                                      tbuf.at[p, u], gsem).wait()
            idv = ids_v[pl.ds(g * _LANES, _LANES)]
            eff = jnp.where(idv < _NUM_USERS, idv, idv - _NUM_USERS)
            sub = eff & 7
            for u in range(_LANES):
                sb = sub[u]
                r = g * _LANES + u
                for cc in range(emb // _LANES):
                    obuf[r, pl.ds(cc * _LANES, _LANES)] = (
                        tbuf[p, u, sb, pl.ds(cc * _LANES, _LANES)])

        fire(0)

        def pipe(g, carry):
            fire(g + 1)
            drain_select(g)
            return carry

        lax.fori_loop(0, ngrp - 1, pipe, 0)
        drain_select(ngrp - 1)
        pltpu.sync_copy(obuf, out_hbm.at[pl.ds(base, bpw)])

    return body, bpw


def kernel(id, e_user, e_item):
    batch = id.shape[0]
    emb = e_user.shape[1]
    info = plsc.get_sparse_core_info()
    nw = info.num_cores * info.num_subcores
    eu3 = e_user.reshape(e_user.shape[0] // 8, 8, emb)
    ei3 = e_item.reshape(e_item.shape[0] // 8, 8, emb)
    body, bpw = _make_body(batch, emb, nw)
    mesh = plsc.VectorSubcoreMesh(core_axis_name="c", subcore_axis_name="s")
    f = pl.kernel(
        body,
        out_type=jax.ShapeDtypeStruct((batch, emb), jnp.float32),
        mesh=mesh,
        compiler_params=pltpu.CompilerParams(use_tc_tiling_on_sc=True),
        scratch_types=[
            pltpu.VMEM((bpw,), jnp.int32),
            pltpu.VMEM((2, _LANES, 8, emb), jnp.float32),
            pltpu.VMEM((bpw, emb), jnp.float32),
            pltpu.SemaphoreType.DMA,
            pltpu.SemaphoreType.DMA,
        ],
    )
    return f(id, eu3, ei3)
